# baseline (speedup 1.0000x reference)
import functools

import jax
import jax.numpy as jnp
from jax import lax
from jax.experimental import pallas as pl
from jax.experimental.pallas import tpu as pltpu
from jax.experimental.pallas import tpu_sc as plsc

N_POINTS = 1048576
EMB_DIM = 32
TABLE_ROWS = 291
TABLE_WORDS = TABLE_ROWS * EMB_DIM

SCALE = (32.0, 128.0, 128.0)
OFFSET = (0.0, 33.0, 162.0)

NC, NS, L = 2, 16, 16
NW = NC * NS
PW = N_POINTS // NW
CHUNK = 256
N_CHUNKS = PW // CHUNK
GROUPS = CHUNK // L
D_VREGS = (TABLE_WORDS - EMB_DIM) // L
IN_W = CHUNK * 3
OUT_W = CHUNK * 96


def _body(inputs_hbm, table_hbm, out_hbm, in_v, t_v, td_v, lo_v, fr_v, out_v,
          sin0, sin1, sout0, sout1):
    sin = (sin0, sin1)
    sout = (sout0, sout1)
    wid = lax.axis_index("s") * NC + lax.axis_index("c")
    base_w = wid * PW

    pltpu.sync_copy(table_hbm, t_v.at[pl.ds(0, TABLE_WORDS)])

    def pack_body(i, carry):
        a = t_v[pl.ds(i * L, L)]
        b = t_v[pl.ds(i * L + EMB_DIM, L)]
        packed = plsc.pack(a, b - a, format=plsc.PackFormat.INTERLEAVED)
        td_v[pl.ds(i * L, L)] = plsc.bitcast(packed, jnp.int32)
        return carry

    lax.fori_loop(0, TABLE_WORDS // L, pack_body, 0, unroll=False)

    def in_copy(ch, b):
        base = base_w + ch * CHUNK
        return tuple(
            pltpu.make_async_copy(
                inputs_hbm.at[pl.ds(a * N_POINTS + base, CHUNK)],
                in_v.at[pl.ds(b * IN_W + a * CHUNK, CHUNK)], sin[b])
            for a in range(3))

    def out_copy(ch, b):
        return pltpu.make_async_copy(
            out_v.at[pl.ds(b * CHUNK, CHUNK), :],
            out_hbm.at[pl.ds(base_w + ch * CHUNK, CHUNK), :], sout[b])

    for c in in_copy(0, 0):
        c.start()
    for c in in_copy(1, 1):
        c.start()

    def chunk2_body(j, carry):
        for b in range(2):
            ch = j * 2 + b
            for c in in_copy(ch, b):
                c.wait()

            def group_body(g, carry2):
                for a in range(3):
                    u = in_v[pl.ds(b * IN_W + a * CHUNK + g * L, L)]
                    c = u * SCALE[a] + OFFSET[a]
                    li = c.astype(jnp.int32)
                    fr = c - li.astype(jnp.float32)
                    lo_v[pl.ds(a * CHUNK + g * L, L)] = li * EMB_DIM
                    fr_v[pl.ds(a * CHUNK + g * L, L)] = fr
                return carry2

            lax.fori_loop(0, GROUPS, group_body, 0, unroll=False)

            @pl.when(ch + 2 < N_CHUNKS)
            def _():
                for c in in_copy(ch + 2, b):
                    c.start()

            @pl.when(ch >= 2)
            def _():
                out_copy(ch - 2, b).wait()

            def pgroup_body(g, carry2):
                lov = [lo_v[pl.ds(a * CHUNK + g * L, L)] for a in range(3)]
                frv = [fr_v[pl.ds(a * CHUNK + g * L, L)] for a in range(3)]
                for i in range(L):
                    prow = b * CHUNK + g * L + i
                    for a in range(3):
                        row = lov[a][i]
                        fr = jnp.full((L,), frv[a][i], dtype=jnp.float32)
                        w0 = td_v[pl.ds(row, L)]
                        w1 = td_v[pl.ds(row + L, L)]
                        t0, d0 = plsc.unpack(
                            plsc.bitcast(w0, jnp.bfloat16),
                            format=plsc.PackFormat.INTERLEAVED,
                            preferred_element_type=jnp.float32)
                        t1, d1 = plsc.unpack(
                            plsc.bitcast(w1, jnp.bfloat16),
                            format=plsc.PackFormat.INTERLEAVED,
                            preferred_element_type=jnp.float32)
                        out_v[prow, pl.ds(a * EMB_DIM, L)] = t0 + fr * d0
                        out_v[prow, pl.ds(a * EMB_DIM + L, L)] = t1 + fr * d1
                return carry2

            lax.fori_loop(0, GROUPS, pgroup_body, 0, unroll=False)

            out_copy(ch, b).start()
        return carry

    lax.fori_loop(0, N_CHUNKS // 2, chunk2_body, 0, unroll=False)

    out_copy(N_CHUNKS - 2, 0).wait()
    out_copy(N_CHUNKS - 1, 1).wait()


@jax.jit
def kernel(inputs, embeddings):
    mesh = plsc.VectorSubcoreMesh(core_axis_name="c", subcore_axis_name="s")
    k = pl.kernel(
        _body,
        out_type=jax.ShapeDtypeStruct((N_POINTS, 96), jnp.float32),
        mesh=mesh,
        compiler_params=pltpu.CompilerParams(needs_layout_passes=False,
                                             use_tc_tiling_on_sc=True),
        scratch_types=[
            pltpu.VMEM((2 * IN_W,), jnp.float32),
            pltpu.VMEM((TABLE_WORDS + 2 * EMB_DIM,), jnp.float32),
            pltpu.VMEM((TABLE_WORDS,), jnp.int32),
            pltpu.VMEM((3 * CHUNK,), jnp.int32),
            pltpu.VMEM((3 * CHUNK,), jnp.float32),
            pltpu.VMEM((2 * CHUNK, 96), jnp.float32),
            pltpu.SemaphoreType.DMA,
            pltpu.SemaphoreType.DMA,
            pltpu.SemaphoreType.DMA,
            pltpu.SemaphoreType.DMA,
        ],
    )
    return k(inputs.T.reshape(-1), embeddings.reshape(-1))


# speedup vs baseline: 1.0567x; 1.0567x over previous
import functools

import jax
import jax.numpy as jnp
from jax import lax
from jax.experimental import pallas as pl
from jax.experimental.pallas import tpu as pltpu
from jax.experimental.pallas import tpu_sc as plsc

N_POINTS = 1048576
EMB_DIM = 32
TABLE_ROWS = 291
TABLE_WORDS = TABLE_ROWS * EMB_DIM

SCALE = (32.0, 128.0, 128.0)
OFFSET = (0.0, 33.0, 162.0)

NC, NS, L = 2, 16, 16
NW = NC * NS
PW = N_POINTS // NW
CHUNK = 256
N_CHUNKS = PW // CHUNK
GROUPS = CHUNK // L
D_VREGS = (TABLE_WORDS - EMB_DIM) // L
IN_W = CHUNK * 3
OUT_W = CHUNK * 96


def _body(inputs_hbm, table_hbm, out_hbm, in_v, t_v, d_v, lo_v, fr_v, out_v,
          sin0, sin1, sout0, sout1):
    sin = (sin0, sin1)
    sout = (sout0, sout1)
    wid = lax.axis_index("s") * NC + lax.axis_index("c")
    base_w = wid * PW

    pltpu.sync_copy(table_hbm, t_v)

    def diff_body(i, carry):
        a = t_v[pl.ds(i * L, L)]
        b = t_v[pl.ds(i * L + EMB_DIM, L)]
        d_v[pl.ds(i * L, L)] = b - a
        return carry

    lax.fori_loop(0, D_VREGS, diff_body, 0, unroll=False)

    def in_copy(ch, b):
        base = base_w + ch * CHUNK
        return tuple(
            pltpu.make_async_copy(
                inputs_hbm.at[pl.ds(a * N_POINTS + base, CHUNK)],
                in_v.at[pl.ds(b * IN_W + a * CHUNK, CHUNK)], sin[b])
            for a in range(3))

    def out_copy(ch, b):
        return pltpu.make_async_copy(
            out_v.at[pl.ds(b * CHUNK, CHUNK), :],
            out_hbm.at[pl.ds(base_w + ch * CHUNK, CHUNK), :], sout[b])

    for c in in_copy(0, 0):
        c.start()
    for c in in_copy(1, 1):
        c.start()

    def chunk2_body(j, carry):
        for b in range(2):
            ch = j * 2 + b
            for c in in_copy(ch, b):
                c.wait()

            def group_body(g, carry2):
                for a in range(3):
                    u = in_v[pl.ds(b * IN_W + a * CHUNK + g * L, L)]
                    c = u * SCALE[a] + OFFSET[a]
                    li = c.astype(jnp.int32)
                    fr = c - li.astype(jnp.float32)
                    lo_v[pl.ds(a * CHUNK + g * L, L)] = li * EMB_DIM
                    fr_v[pl.ds(a * CHUNK + g * L, L)] = fr
                return carry2

            lax.fori_loop(0, GROUPS, group_body, 0, unroll=False)

            @pl.when(ch + 2 < N_CHUNKS)
            def _():
                for c in in_copy(ch + 2, b):
                    c.start()

            @pl.when(ch >= 2)
            def _():
                out_copy(ch - 2, b).wait()

            iota16 = lax.iota(jnp.int32, L)
            iota16b = iota16 + L

            def pgroup_body(g, carry2):
                lov = [lo_v[pl.ds(a * CHUNK + g * L, L)] for a in range(3)]
                frv = [fr_v[pl.ds(a * CHUNK + g * L, L)] for a in range(3)]
                for i in range(L):
                    prow = b * CHUNK + g * L + i
                    for a in range(3):
                        rowv = jnp.full((L,), lov[a][i], dtype=jnp.int32)
                        fr = jnp.full((L,), frv[a][i], dtype=jnp.float32)
                        i0 = rowv + iota16
                        i1 = rowv + iota16b
                        t0 = plsc.load_gather(t_v, [i0])
                        t1 = plsc.load_gather(t_v, [i1])
                        d0 = plsc.load_gather(d_v, [i0])
                        d1 = plsc.load_gather(d_v, [i1])
                        out_v[prow, pl.ds(a * EMB_DIM, L)] = t0 + fr * d0
                        out_v[prow, pl.ds(a * EMB_DIM + L, L)] = t1 + fr * d1
                return carry2

            lax.fori_loop(0, GROUPS, pgroup_body, 0, unroll=False)

            out_copy(ch, b).start()
        return carry

    lax.fori_loop(0, N_CHUNKS // 2, chunk2_body, 0, unroll=False)

    out_copy(N_CHUNKS - 2, 0).wait()
    out_copy(N_CHUNKS - 1, 1).wait()


@jax.jit
def kernel(inputs, embeddings):
    mesh = plsc.VectorSubcoreMesh(core_axis_name="c", subcore_axis_name="s")
    k = pl.kernel(
        _body,
        out_type=jax.ShapeDtypeStruct((N_POINTS, 96), jnp.float32),
        mesh=mesh,
        compiler_params=pltpu.CompilerParams(needs_layout_passes=False,
                                             use_tc_tiling_on_sc=True),
        scratch_types=[
            pltpu.VMEM((2 * IN_W,), jnp.float32),
            pltpu.VMEM((TABLE_WORDS,), jnp.float32),
            pltpu.VMEM((TABLE_WORDS,), jnp.float32),
            pltpu.VMEM((3 * CHUNK,), jnp.int32),
            pltpu.VMEM((3 * CHUNK,), jnp.float32),
            pltpu.VMEM((2 * CHUNK, 96), jnp.float32),
            pltpu.SemaphoreType.DMA,
            pltpu.SemaphoreType.DMA,
            pltpu.SemaphoreType.DMA,
            pltpu.SemaphoreType.DMA,
        ],
    )
    return k(inputs.T.reshape(-1), embeddings.reshape(-1))
